# Initial kernel scaffold; baseline (speedup 1.0000x reference)
import jax
import jax.numpy as jnp
from jax.experimental import pallas as pl


def kernel(positions, bucket_embed, W1, b1, W2, b2, Wc, bc):
    raise NotImplementedError("write your pallas kernel here")


# speedup vs baseline: 3.6946x; 3.6946x over previous
import functools

import jax
import jax.numpy as jnp
from jax.experimental import pallas as pl
from jax.experimental.pallas import tpu as pltpu

N_EDGES = 320000
DIM = 128
NUM_BUCKETS = 32
EDGE_BLOCK = 4000


def _fused_kernel(pos_ref, be_ref, w1t_ref, b1r_ref, w2t_ref, b2r_ref,
                  wct_ref, bcr_ref, out_ref, t_s, m_s, bias_s):
    @pl.when(pl.program_id(0) == 0)
    def _fold():
        a = wct_ref[0:DIM, :]
        b = wct_ref[DIM:2 * DIM, :]
        t_s[:] = jnp.dot(be_ref[:], a, preferred_element_type=jnp.float32)
        m_s[:] = jnp.dot(w2t_ref[:], b, preferred_element_type=jnp.float32)
        bias_s[:] = jnp.dot(b2r_ref[:], b,
                            preferred_element_type=jnp.float32) + bcr_ref[:]

    p = pos_ref[:]
    s = p * w1t_ref[:] + b1r_ref[:]
    h = s * jax.nn.sigmoid(s)
    idx = jnp.clip((p * (NUM_BUCKETS - 1)).astype(jnp.int32), 0,
                   NUM_BUCKETS - 1)
    lanes = jax.lax.broadcasted_iota(jnp.int32, (p.shape[0], NUM_BUCKETS), 1)
    oh = (lanes == idx).astype(jnp.float32)
    out_ref[:] = (jnp.dot(oh, t_s[:], preferred_element_type=jnp.float32) +
                  jnp.dot(h, m_s[:], preferred_element_type=jnp.float32) +
                  bias_s[:])


@jax.jit
def kernel(positions, bucket_embed, W1, b1, W2, b2, Wc, bc):
    n = positions.shape[0]
    pos2d = positions.reshape(n, 1)
    w1t = W1.reshape(1, DIM)
    b1r = b1.reshape(1, DIM)
    w2t = W2.T
    b2r = b2.reshape(1, DIM)
    wct = Wc.T
    bcr = bc.reshape(1, DIM)

    grid = n // EDGE_BLOCK
    out = pl.pallas_call(
        _fused_kernel,
        grid=(grid,),
        in_specs=[
            pl.BlockSpec((EDGE_BLOCK, 1), lambda g: (g, 0)),
            pl.BlockSpec((NUM_BUCKETS, DIM), lambda g: (0, 0)),
            pl.BlockSpec((1, DIM), lambda g: (0, 0)),
            pl.BlockSpec((1, DIM), lambda g: (0, 0)),
            pl.BlockSpec((DIM, DIM), lambda g: (0, 0)),
            pl.BlockSpec((1, DIM), lambda g: (0, 0)),
            pl.BlockSpec((2 * DIM, DIM), lambda g: (0, 0)),
            pl.BlockSpec((1, DIM), lambda g: (0, 0)),
        ],
        out_specs=pl.BlockSpec((EDGE_BLOCK, DIM), lambda g: (g, 0)),
        out_shape=jax.ShapeDtypeStruct((n, DIM), jnp.float32),
        scratch_shapes=[
            pltpu.VMEM((NUM_BUCKETS, DIM), jnp.float32),
            pltpu.VMEM((DIM, DIM), jnp.float32),
            pltpu.VMEM((1, DIM), jnp.float32),
        ],
        compiler_params=pltpu.CompilerParams(
            dimension_semantics=("arbitrary",)),
    )(pos2d, bucket_embed, w1t, b1r, w2t, b2r, wct, bcr)
    return out
